# baseline (speedup 1.0000x reference)
import functools

import jax
import jax.numpy as jnp
from jax import lax
from jax.experimental import pallas as pl
from jax.experimental.pallas import tpu as pltpu
from jax.experimental.pallas import tpu_sc as plsc

N = 10000
E = 320000
D = 128
ALPHA = 0.2

NC = 2
NS = 16
EPC = E // NC
C = 160
NCHUNK = EPC // C
RPT = 624
RPT_LAST = N - RPT * (NS - 1)
GSUB = 32
RSROWS = RPT_LAST // 8
ACC1 = RPT_LAST + RSROWS
ACC2 = RPT_LAST + 8


def _pre_body(x_ref, w_ref, ac_ref, h_ref, s_ref):
    h = jnp.dot(x_ref[...], w_ref[...], preferred_element_type=jnp.float32)
    h_ref[...] = h
    s_ref[...] = jnp.dot(h, ac_ref[...], preferred_element_type=jnp.float32)


def _tile_bounds(sid):
    lo = sid * RPT
    nr = jnp.where(sid == NS - 1, RPT_LAST, RPT)
    return lo, nr


def _zero_acc(acc_ref, nrows):
    zero16 = jnp.zeros((16,), jnp.float32)

    def zrow(i, carry):
        for q in range(D // 16):
            acc_ref[i, pl.ds(q * 16, 16)] = zero16
        return carry

    lax.fori_loop(0, nrows, zrow, 0)


def _zero_idx(idx_ref):
    zero16 = jnp.zeros((16,), jnp.int32)
    for j in range(C // 16):
        idx_ref[pl.ds(j * 16, 16)] = zero16


def _scan_compact(src_v, dst_v, srcc_v, dstc_v, lo, nr):
    off = jnp.int32(0)
    lo16 = jnp.broadcast_to(lo, (16,))
    hi16 = jnp.broadcast_to(lo + nr, (16,))
    for j in range(C // 16):
        s16 = src_v[pl.ds(j * 16, 16)]
        d16 = dst_v[pl.ds(j * 16, 16)]
        mask = jnp.logical_and(s16 >= lo16, s16 < hi16)
        plsc.store_compressed(srcc_v.at[pl.ds(off, 16)], s16, mask=mask)
        plsc.store_compressed(dstc_v.at[pl.ds(off, 16)], d16, mask=mask)
        off = off + jnp.sum(mask.astype(jnp.int32))
    return off


def _edge1_body(src_hbm, dst_hbm, h_hbm, s1_hbm, s2_hbm, hp_out, rs_out,
                src_v, dst_v, srcc_v, dstc_v, ev_v, slc_v,
                rows_v, s1loc_v, s2_v, acc_v, gsem):
    cid = lax.axis_index("c")
    sid = lax.axis_index("s")
    lo, nr = _tile_bounds(sid)
    iota16 = lax.iota(jnp.int32, 16)

    _zero_acc(acc_v, ACC1)
    _zero_idx(srcc_v)
    _zero_idx(dstc_v)
    pltpu.sync_copy(s2_hbm, s2_v)
    pltpu.sync_copy(s1_hbm.at[pl.ds(lo, RPT_LAST)], s1loc_v)

    def chunk(k, carry):
        base = cid * EPC + k * C
        pltpu.sync_copy(src_hbm.at[pl.ds(base, C)], src_v)
        pltpu.sync_copy(dst_hbm.at[pl.ds(base, C)], dst_v)

        n_k = _scan_compact(src_v, dst_v, srcc_v, dstc_v, lo, nr)

        @pl.when(n_k > 0)
        def _():
            cps = [
                pltpu.async_copy(h_hbm.at[dstc_v.at[pl.ds(b * GSUB, GSUB)]],
                                 rows_v.at[pl.ds(b * GSUB, GSUB)], gsem)
                for b in range(C // GSUB)
            ]

            n_g = (n_k + 15) // 16

            def prep(g, carry2):
                valid = (g * 16 + iota16) < n_k
                sl16 = jnp.where(
                    valid, srcc_v[pl.ds(g * 16, 16)] - lo, 0)
                sval = plsc.load_gather(s1loc_v, [sl16])
                dval = plsc.load_gather(s2_v, [dstc_v[pl.ds(g * 16, 16)]])
                t = sval + dval
                lr = jnp.where(t > 0.0, t, ALPHA * t)
                e = jnp.exp(-lr)
                ev_v[pl.ds(g * 16, 16)] = jnp.where(valid, e, 0.0)
                slc_v[pl.ds(g * 16, 16)] = sl16
                return carry2

            for cp in cps:
                cp.wait()

            def accum(g, carry2):
                sl16 = slc_v[pl.ds(g * 16, 16)]
                e16 = ev_v[pl.ds(g * 16, 16)]
                for l in range(16):
                    sl = sl16[l]
                    e = e16[l]
                    i = g * 16 + l
                    for q in range(D // 16):
                        acc_v[sl, pl.ds(q * 16, 16)] = (
                            acc_v[sl, pl.ds(q * 16, 16)]
                            + e * rows_v[i, pl.ds(q * 16, 16)])
                    rrow = RPT_LAST + (sl >> 3)
                    rcol = (sl & 7) * 16
                    acc_v[rrow, pl.ds(rcol, 16)] = (
                        acc_v[rrow, pl.ds(rcol, 16)] + e)
                return carry2

        return carry

    lax.fori_loop(0, NCHUNK, chunk, 0)

    @pl.when(sid < NS - 1)
    def _():
        pltpu.sync_copy(acc_v.at[pl.ds(0, RPT)],
                        hp_out.at[pl.ds(cid * N + lo, RPT)])

    @pl.when(sid == NS - 1)
    def _():
        pltpu.sync_copy(acc_v.at[pl.ds(0, RPT_LAST)],
                        hp_out.at[pl.ds(cid * N + lo, RPT_LAST)])

    wid = cid * NS + sid
    pltpu.sync_copy(acc_v.at[pl.ds(RPT_LAST, RSROWS)],
                    rs_out.at[pl.ds(wid * RSROWS, RSROWS)])


def _mid_body(hp_ref, rs_ref, x_ref, m_ref, se_ref):
    hp = hp_ref[0] + hp_ref[1]
    rs = rs_ref[...].sum(axis=1, keepdims=True)
    hprime = hp / (rs + 1e-16)
    m0 = jnp.where(hprime > 0.0, hprime, jnp.exp(hprime) - 1.0)
    mx = jnp.max(m0, axis=0, keepdims=True)
    z = jnp.exp(m0 - mx)
    sm = jnp.sum(z, axis=0, keepdims=True)
    m = z / sm
    m_ref[...] = m
    se_ref[...] = lax.dot_general(m, x_ref[...], (((0,), (0,)), ((), ())),
                                  preferred_element_type=jnp.float32)


def _edge2_body(src_hbm, dst_hbm, m_hbm, si_out,
                src_v, dst_v, srcc_v, dstc_v, slc_v, rows_v, acc_v, gsem):
    cid = lax.axis_index("c")
    sid = lax.axis_index("s")
    lo, nr = _tile_bounds(sid)
    iota16 = lax.iota(jnp.int32, 16)

    _zero_acc(acc_v, ACC2)
    _zero_idx(srcc_v)
    _zero_idx(dstc_v)

    def chunk(k, carry):
        base = cid * EPC + k * C
        pltpu.sync_copy(src_hbm.at[pl.ds(base, C)], src_v)
        pltpu.sync_copy(dst_hbm.at[pl.ds(base, C)], dst_v)

        n_k = _scan_compact(src_v, dst_v, srcc_v, dstc_v, lo, nr)

        @pl.when(n_k > 0)
        def _():
            cps = [
                pltpu.async_copy(m_hbm.at[dstc_v.at[pl.ds(b * GSUB, GSUB)]],
                                 rows_v.at[pl.ds(b * GSUB, GSUB)], gsem)
                for b in range(C // GSUB)
            ]

            n_g = (n_k + 15) // 16

            def prep(g, carry2):
                valid = (g * 16 + iota16) < n_k
                slc_v[pl.ds(g * 16, 16)] = jnp.where(
                    valid, srcc_v[pl.ds(g * 16, 16)] - lo, RPT_LAST)
                return carry2

            lax.fori_loop(0, n_g, prep, 0)
            for cp in cps:
                cp.wait()

            def accum(g, carry2):
                sl16 = slc_v[pl.ds(g * 16, 16)]
                for l in range(16):
                    sl = sl16[l]
                    i = g * 16 + l
                    for q in range(D // 16):
                        acc_v[sl, pl.ds(q * 16, 16)] = (
                            acc_v[sl, pl.ds(q * 16, 16)]
                            + rows_v[i, pl.ds(q * 16, 16)])
                return carry2

        return carry

    lax.fori_loop(0, NCHUNK, chunk, 0)

    @pl.when(sid < NS - 1)
    def _():
        pltpu.sync_copy(acc_v.at[pl.ds(0, RPT)],
                        si_out.at[pl.ds(cid * N + lo, RPT)])

    @pl.when(sid == NS - 1)
    def _():
        pltpu.sync_copy(acc_v.at[pl.ds(0, RPT_LAST)],
                        si_out.at[pl.ds(cid * N + lo, RPT_LAST)])


def _post_body(si_ref, m_ref, sa_ref):
    si = si_ref[0] + si_ref[1]
    t = lax.dot_general(m_ref[...], si, (((0,), (0,)), ((), ())),
                        preferred_element_type=jnp.float32)
    sa_ref[...] = jnp.maximum(t - 1e-4, 0.0)


def kernel(main_feat, edge_index, W, a):
    f32 = jnp.float32
    src = edge_index[0]
    dst = edge_index[1]
    acols = a[0].reshape(2, D).T

    h, s = pl.pallas_call(
        _pre_body,
        out_shape=[jax.ShapeDtypeStruct((N, D), f32),
                   jax.ShapeDtypeStruct((N, 2), f32)],
    )(main_feat, W, acols)
    s1 = s[:, 0]
    s2 = s[:, 1]

    mesh = plsc.VectorSubcoreMesh(core_axis_name="c", subcore_axis_name="s")
    scp = pltpu.CompilerParams(needs_layout_passes=False)
    edge1 = pl.kernel(
        _edge1_body,
        out_type=[jax.ShapeDtypeStruct((NC * N, D), f32),
                  jax.ShapeDtypeStruct((NC * NS * RSROWS, D), f32)],
        mesh=mesh,
        compiler_params=scp,
        scratch_types=[
            pltpu.VMEM((C,), jnp.int32),
            pltpu.VMEM((C,), jnp.int32),
            pltpu.VMEM((C,), jnp.int32),
            pltpu.VMEM((C,), jnp.int32),
            pltpu.VMEM((C,), f32),
            pltpu.VMEM((C,), jnp.int32),
            pltpu.VMEM((C, D), f32),
            pltpu.VMEM((RPT_LAST,), f32),
            pltpu.VMEM((N,), f32),
            pltpu.VMEM((ACC1, D), f32),
            pltpu.SemaphoreType.DMA,
        ],
    )
    hp2, rs2 = edge1(src, dst, h, s1, s2)
    hp = hp2.reshape(NC, N, D)

    rs4 = rs2.reshape(NC, NS, RSROWS * 8, 16)[:, :, :, 0]
    parts = [rs4[:, t, :RPT] for t in range(NS - 1)] + [rs4[:, NS - 1, :]]
    rs = jnp.concatenate(parts, axis=1).T

    m, struct_emb = pl.pallas_call(
        _mid_body,
        out_shape=[jax.ShapeDtypeStruct((N, D), f32),
                   jax.ShapeDtypeStruct((D, D), f32)],
    )(hp, rs, main_feat)

    edge2 = pl.kernel(
        _edge2_body,
        out_type=jax.ShapeDtypeStruct((NC * N, D), f32),
        mesh=mesh,
        compiler_params=scp,
        scratch_types=[
            pltpu.VMEM((C,), jnp.int32),
            pltpu.VMEM((C,), jnp.int32),
            pltpu.VMEM((C,), jnp.int32),
            pltpu.VMEM((C,), jnp.int32),
            pltpu.VMEM((C,), jnp.int32),
            pltpu.VMEM((C, D), f32),
            pltpu.VMEM((ACC2, D), f32),
            pltpu.SemaphoreType.DMA,
        ],
    )
    si2 = edge2(src, dst, m)
    si = si2.reshape(NC, N, D)

    struct_adj = pl.pallas_call(
        _post_body,
        out_shape=jax.ShapeDtypeStruct((D, D), f32),
    )(si, m)

    return (struct_emb, struct_adj, m)


# speedup vs baseline: 176.4759x; 176.4759x over previous
import functools

import jax
import jax.numpy as jnp
from jax import lax
from jax.experimental import pallas as pl
from jax.experimental.pallas import tpu as pltpu
from jax.experimental.pallas import tpu_sc as plsc

N = 10000
E = 320000
D = 128
ALPHA = 0.2

NC = 2
NS = 16
NW = NC * NS
EPC = E // NC
CS = 640
NCHUNK = EPC // CS
FC = 160
SPILL = CS + FC + 16
CAP = EPC + FC
RPT = 624
RPT_LAST = N - RPT * (NS - 1)
JUNK = 720
RSROWS = RPT_LAST // 8
ACC1 = 736
ACC2 = 728


def _pre_body(x_ref, w_ref, ac_ref, h_ref, s_ref):
    h = jnp.dot(x_ref[...], w_ref[...], preferred_element_type=jnp.float32)
    h_ref[...] = h
    s_ref[...] = jnp.dot(h, ac_ref[...], preferred_element_type=jnp.float32)


def _tile_bounds(sid):
    lo = sid * RPT
    nr = jnp.where(sid == NS - 1, RPT_LAST, RPT)
    return lo, nr


def _scan_body(src_hbm, dst_hbm, s1_hbm, s2_hbm, slc_hbm, dstc_hbm, ev_hbm,
               cnt_hbm, src_v, dst_v, slb_v, dsb_v, evb_v, s1loc_v, s2_v,
               cnt_v):
    cid = lax.axis_index("c")
    sid = lax.axis_index("s")
    lo, nr = _tile_bounds(sid)
    wid = cid * NS + sid
    obase = wid * CAP
    lo16 = jnp.broadcast_to(lo, (16,))
    hi16 = jnp.broadcast_to(lo + nr, (16,))

    pltpu.sync_copy(s2_hbm, s2_v)
    pltpu.sync_copy(s1_hbm.at[pl.ds(lo, RPT_LAST)], s1loc_v)

    def flush(nf):
        for g in range(FC // 16):
            sl16 = jnp.minimum(slb_v[pl.ds(g * 16, 16)], RPT_LAST - 1)
            d16 = dsb_v[pl.ds(g * 16, 16)]
            sval = plsc.load_gather(s1loc_v, [sl16])
            dval = plsc.load_gather(s2_v, [d16])
            t = sval + dval
            lr = jnp.where(t > 0.0, t, ALPHA * t)
            evb_v[pl.ds(g * 16, 16)] = jnp.exp(-lr)
        out = obase + nf * FC
        pltpu.sync_copy(slb_v.at[pl.ds(0, FC)], slc_hbm.at[pl.ds(out, FC)])
        pltpu.sync_copy(dsb_v.at[pl.ds(0, FC)], dstc_hbm.at[pl.ds(out, FC)])
        pltpu.sync_copy(evb_v.at[pl.ds(0, FC)], ev_hbm.at[pl.ds(out, FC)])
        for g in range((SPILL - FC) // 16):
            slb_v[pl.ds(g * 16, 16)] = slb_v[pl.ds(FC + g * 16, 16)]
            dsb_v[pl.ds(g * 16, 16)] = dsb_v[pl.ds(FC + g * 16, 16)]

    def chunk(k, carry):
        ln, nf = carry
        base = cid * EPC + k * CS
        pltpu.sync_copy(src_hbm.at[pl.ds(base, CS)], src_v)
        pltpu.sync_copy(dst_hbm.at[pl.ds(base, CS)], dst_v)

        for j in range(CS // 16):
            s16 = src_v[pl.ds(j * 16, 16)]
            d16 = dst_v[pl.ds(j * 16, 16)]
            mask = jnp.logical_and(s16 >= lo16, s16 < hi16)
            plsc.store_compressed(slb_v.at[pl.ds(ln, 16)], s16 - lo16,
                                  mask=mask)
            plsc.store_compressed(dsb_v.at[pl.ds(ln, 16)], d16, mask=mask)
            ln = ln + jnp.sum(mask.astype(jnp.int32))

        for _ in range(CS // FC):
            @pl.when(ln >= FC)
            def _():
                flush(nf)

            nf = jnp.where(ln >= FC, nf + 1, nf)
            ln = jnp.where(ln >= FC, ln - FC, ln)
        return (ln, nf)

    ln, nf = lax.fori_loop(0, NCHUNK, chunk, (jnp.int32(0), jnp.int32(0)))

    @pl.when(ln > 0)
    def _():
        lnc = ln
        junk16 = jnp.full((16,), JUNK, jnp.int32)
        zero16 = jnp.zeros((16,), jnp.int32)
        iota16 = lax.iota(jnp.int32, 16)
        for g in range(FC // 16):
            cur = slb_v[pl.ds(g * 16, 16)]
            curd = dsb_v[pl.ds(g * 16, 16)]
            valid = (g * 16 + iota16) < lnc
            slb_v[pl.ds(g * 16, 16)] = jnp.where(valid, cur, junk16)
            dsb_v[pl.ds(g * 16, 16)] = jnp.where(valid, curd, zero16)
        flush(nf)

    nf = jnp.where(ln > 0, nf + 1, nf)
    cnt_v[pl.ds(0, 16)] = jnp.broadcast_to(nf, (16,))
    pltpu.sync_copy(cnt_v, cnt_hbm.at[pl.ds(wid * 16, 16)])


def _acc1_body(slc_hbm, dstc_hbm, ev_hbm, cnt_hbm, h_hbm, hp_out, rs_out,
               slc_v, dstc_v, ev_v, cnt_v, rows_v, acc_v, gsem):
    cid = lax.axis_index("c")
    sid = lax.axis_index("s")
    lo, nr = _tile_bounds(sid)
    wid = cid * NS + sid
    obase = wid * CAP

    zero16 = jnp.zeros((16,), jnp.float32)

    def zrow(i, carry):
        for q in range(D // 16):
            acc_v[i, pl.ds(q * 16, 16)] = zero16
        return carry

    lax.fori_loop(0, ACC1, zrow, 0)

    pltpu.sync_copy(cnt_hbm.at[pl.ds(wid * 16, 16)], cnt_v)
    n_blocks = cnt_v[pl.ds(0, 16)][0]

    def block(k, carry):
        base = obase + k * FC
        pltpu.sync_copy(slc_hbm.at[pl.ds(base, FC)], slc_v)
        pltpu.sync_copy(dstc_hbm.at[pl.ds(base, FC)], dstc_v)
        pltpu.sync_copy(ev_hbm.at[pl.ds(base, FC)], ev_v)
        pltpu.async_copy(h_hbm.at[dstc_v], rows_v, gsem).wait()

        def accum(g, carry2):
            sl16 = slc_v[pl.ds(g * 16, 16)]
            e16 = ev_v[pl.ds(g * 16, 16)]
            for l in range(16):
                sl = sl16[l]
                e = e16[l]
                i = g * 16 + l
                for q in range(D // 16):
                    acc_v[sl, pl.ds(q * 16, 16)] = (
                        acc_v[sl, pl.ds(q * 16, 16)]
                        + e * rows_v[i, pl.ds(q * 16, 16)])
                rrow = RPT_LAST + (sl >> 3)
                rcol = (sl & 7) * 16
                acc_v[rrow, pl.ds(rcol, 16)] = (
                    acc_v[rrow, pl.ds(rcol, 16)] + e)
            return carry2

        lax.fori_loop(0, FC // 16, accum, 0)
        return carry

    lax.fori_loop(0, n_blocks, block, 0)

    @pl.when(sid < NS - 1)
    def _():
        pltpu.sync_copy(acc_v.at[pl.ds(0, RPT)],
                        hp_out.at[pl.ds(cid * N + lo, RPT)])

    @pl.when(sid == NS - 1)
    def _():
        pltpu.sync_copy(acc_v.at[pl.ds(0, RPT_LAST)],
                        hp_out.at[pl.ds(cid * N + lo, RPT_LAST)])

    pltpu.sync_copy(acc_v.at[pl.ds(RPT_LAST, RSROWS)],
                    rs_out.at[pl.ds(wid * RSROWS, RSROWS)])


def _acc2_body(slc_hbm, dstc_hbm, cnt_hbm, m_hbm, si_out,
               slc_v, dstc_v, cnt_v, rows_v, acc_v, gsem):
    cid = lax.axis_index("c")
    sid = lax.axis_index("s")
    lo, nr = _tile_bounds(sid)
    wid = cid * NS + sid
    obase = wid * CAP

    zero16 = jnp.zeros((16,), jnp.float32)

    def zrow(i, carry):
        for q in range(D // 16):
            acc_v[i, pl.ds(q * 16, 16)] = zero16
        return carry

    lax.fori_loop(0, ACC2, zrow, 0)

    pltpu.sync_copy(cnt_hbm.at[pl.ds(wid * 16, 16)], cnt_v)
    n_blocks = cnt_v[pl.ds(0, 16)][0]

    def block(k, carry):
        base = obase + k * FC
        pltpu.sync_copy(slc_hbm.at[pl.ds(base, FC)], slc_v)
        pltpu.sync_copy(dstc_hbm.at[pl.ds(base, FC)], dstc_v)
        pltpu.async_copy(m_hbm.at[dstc_v], rows_v, gsem).wait()

        def accum(g, carry2):
            sl16 = slc_v[pl.ds(g * 16, 16)]
            for l in range(16):
                sl = sl16[l]
                i = g * 16 + l
                for q in range(D // 16):
                    acc_v[sl, pl.ds(q * 16, 16)] = (
                        acc_v[sl, pl.ds(q * 16, 16)]
                        + rows_v[i, pl.ds(q * 16, 16)])
            return carry2

        lax.fori_loop(0, FC // 16, accum, 0)
        return carry

    lax.fori_loop(0, n_blocks, block, 0)

    @pl.when(sid < NS - 1)
    def _():
        pltpu.sync_copy(acc_v.at[pl.ds(0, RPT)],
                        si_out.at[pl.ds(cid * N + lo, RPT)])

    @pl.when(sid == NS - 1)
    def _():
        pltpu.sync_copy(acc_v.at[pl.ds(0, RPT_LAST)],
                        si_out.at[pl.ds(cid * N + lo, RPT_LAST)])


def _mid_body(hp_ref, rs_ref, x_ref, m_ref, se_ref):
    hp = hp_ref[0] + hp_ref[1]
    rs = rs_ref[...].sum(axis=1, keepdims=True)
    hprime = hp / (rs + 1e-16)
    m0 = jnp.where(hprime > 0.0, hprime, jnp.exp(hprime) - 1.0)
    mx = jnp.max(m0, axis=0, keepdims=True)
    z = jnp.exp(m0 - mx)
    sm = jnp.sum(z, axis=0, keepdims=True)
    m = z / sm
    m_ref[...] = m
    se_ref[...] = lax.dot_general(m, x_ref[...], (((0,), (0,)), ((), ())),
                                  preferred_element_type=jnp.float32)


def _post_body(si_ref, m_ref, sa_ref):
    si = si_ref[0] + si_ref[1]
    t = lax.dot_general(m_ref[...], si, (((0,), (0,)), ((), ())),
                        preferred_element_type=jnp.float32)
    sa_ref[...] = jnp.maximum(t - 1e-4, 0.0)


def kernel(main_feat, edge_index, W, a):
    f32 = jnp.float32
    i32 = jnp.int32
    src = edge_index[0]
    dst = edge_index[1]
    acols = a[0].reshape(2, D).T

    h, s = pl.pallas_call(
        _pre_body,
        out_shape=[jax.ShapeDtypeStruct((N, D), f32),
                   jax.ShapeDtypeStruct((N, 2), f32)],
    )(main_feat, W, acols)
    s1 = s[:, 0]
    s2 = s[:, 1]

    mesh = plsc.VectorSubcoreMesh(core_axis_name="c", subcore_axis_name="s")

    scan = pl.kernel(
        _scan_body,
        out_type=[jax.ShapeDtypeStruct((NW * CAP,), i32),
                  jax.ShapeDtypeStruct((NW * CAP,), i32),
                  jax.ShapeDtypeStruct((NW * CAP,), f32),
                  jax.ShapeDtypeStruct((NW * 16,), i32)],
        mesh=mesh,
        compiler_params=pltpu.CompilerParams(needs_layout_passes=False),
        scratch_types=[
            pltpu.VMEM((CS,), i32),
            pltpu.VMEM((CS,), i32),
            pltpu.VMEM((SPILL,), i32),
            pltpu.VMEM((SPILL,), i32),
            pltpu.VMEM((FC,), f32),
            pltpu.VMEM((RPT_LAST,), f32),
            pltpu.VMEM((N,), f32),
            pltpu.VMEM((16,), i32),
        ],
    )
    slc, dstc, ev, cnt = scan(src, dst, s1, s2)

    acc1 = pl.kernel(
        _acc1_body,
        out_type=[jax.ShapeDtypeStruct((NC * N, D), f32),
                  jax.ShapeDtypeStruct((NW * RSROWS, D), f32)],
        mesh=mesh,
        scratch_types=[
            pltpu.VMEM((FC,), i32),
            pltpu.VMEM((FC,), i32),
            pltpu.VMEM((FC,), f32),
            pltpu.VMEM((16,), i32),
            pltpu.VMEM((FC, D), f32),
            pltpu.VMEM((ACC1, D), f32),
            pltpu.SemaphoreType.DMA,
        ],
    )
    hp2, rs2 = acc1(slc, dstc, ev, cnt, h)
    hp = hp2.reshape(NC, N, D)

    rs4 = rs2.reshape(NC, NS, RSROWS * 8, 16)[:, :, :, 0]
    parts = [rs4[:, t, :RPT] for t in range(NS - 1)] + [rs4[:, NS - 1, :]]
    rs = jnp.concatenate(parts, axis=1).T

    m, struct_emb = pl.pallas_call(
        _mid_body,
        out_shape=[jax.ShapeDtypeStruct((N, D), f32),
                   jax.ShapeDtypeStruct((D, D), f32)],
    )(hp, rs, main_feat)

    acc2 = pl.kernel(
        _acc2_body,
        out_type=jax.ShapeDtypeStruct((NC * N, D), f32),
        mesh=mesh,
        scratch_types=[
            pltpu.VMEM((FC,), i32),
            pltpu.VMEM((FC,), i32),
            pltpu.VMEM((16,), i32),
            pltpu.VMEM((FC, D), f32),
            pltpu.VMEM((ACC2, D), f32),
            pltpu.SemaphoreType.DMA,
        ],
    )
    si2 = acc2(slc, dstc, cnt, m)
    si = si2.reshape(NC, N, D)

    struct_adj = pl.pallas_call(
        _post_body,
        out_shape=jax.ShapeDtypeStruct((D, D), f32),
    )(si, m)

    return (struct_emb, struct_adj, m)


# speedup vs baseline: 176.5231x; 1.0003x over previous
import functools

import jax
import jax.numpy as jnp
from jax import lax
from jax.experimental import pallas as pl
from jax.experimental.pallas import tpu as pltpu
from jax.experimental.pallas import tpu_sc as plsc

N = 10000
E = 320000
D = 128
ALPHA = 0.2

NC = 2
NS = 16
NW = NC * NS
EPC = E // NC
CS = 640
NCHUNK = EPC // CS
FC = 160
GSUB = 40
SPILL = CS + FC + 16
CAP = EPC + FC
RPT = 624
RPT_LAST = N - RPT * (NS - 1)
JUNK = 720
RSROWS = RPT_LAST // 8
ACC1 = 736
ACC2 = 728


def _pre_body(x_ref, w_ref, ac_ref, h_ref, s_ref):
    h = jnp.dot(x_ref[...], w_ref[...], preferred_element_type=jnp.float32)
    h_ref[...] = h
    s_ref[...] = jnp.dot(h, ac_ref[...], preferred_element_type=jnp.float32)


def _tile_bounds(sid):
    lo = sid * RPT
    nr = jnp.where(sid == NS - 1, RPT_LAST, RPT)
    return lo, nr


def _scan_body(src_hbm, dst_hbm, s1_hbm, s2_hbm, slc_hbm, dstc_hbm, ev_hbm,
               cnt_hbm, src_v, dst_v, slb_v, dsb_v, evb_v, s1loc_v, s2_v,
               cnt_v):
    cid = lax.axis_index("c")
    sid = lax.axis_index("s")
    lo, nr = _tile_bounds(sid)
    wid = cid * NS + sid
    obase = wid * CAP
    lo16 = jnp.broadcast_to(lo, (16,))
    hi16 = jnp.broadcast_to(lo + nr, (16,))

    pltpu.sync_copy(s2_hbm, s2_v)
    pltpu.sync_copy(s1_hbm.at[pl.ds(lo, RPT_LAST)], s1loc_v)

    def flush(nf):
        for g in range(FC // 16):
            sl16 = jnp.minimum(slb_v[pl.ds(g * 16, 16)], RPT_LAST - 1)
            d16 = dsb_v[pl.ds(g * 16, 16)]
            sval = plsc.load_gather(s1loc_v, [sl16])
            dval = plsc.load_gather(s2_v, [d16])
            t = sval + dval
            lr = jnp.where(t > 0.0, t, ALPHA * t)
            evb_v[pl.ds(g * 16, 16)] = jnp.exp(-lr)
        out = obase + nf * FC
        pltpu.sync_copy(slb_v.at[pl.ds(0, FC)], slc_hbm.at[pl.ds(out, FC)])
        pltpu.sync_copy(dsb_v.at[pl.ds(0, FC)], dstc_hbm.at[pl.ds(out, FC)])
        pltpu.sync_copy(evb_v.at[pl.ds(0, FC)], ev_hbm.at[pl.ds(out, FC)])
        for g in range((SPILL - FC) // 16):
            slb_v[pl.ds(g * 16, 16)] = slb_v[pl.ds(FC + g * 16, 16)]
            dsb_v[pl.ds(g * 16, 16)] = dsb_v[pl.ds(FC + g * 16, 16)]

    def chunk(k, carry):
        ln, nf = carry
        base = cid * EPC + k * CS
        pltpu.sync_copy(src_hbm.at[pl.ds(base, CS)], src_v)
        pltpu.sync_copy(dst_hbm.at[pl.ds(base, CS)], dst_v)

        for j in range(CS // 16):
            s16 = src_v[pl.ds(j * 16, 16)]
            d16 = dst_v[pl.ds(j * 16, 16)]
            mask = jnp.logical_and(s16 >= lo16, s16 < hi16)
            plsc.store_compressed(slb_v.at[pl.ds(ln, 16)], s16 - lo16,
                                  mask=mask)
            plsc.store_compressed(dsb_v.at[pl.ds(ln, 16)], d16, mask=mask)
            ln = ln + jnp.sum(mask.astype(jnp.int32))

        for _ in range(CS // FC):
            @pl.when(ln >= FC)
            def _():
                flush(nf)

            nf = jnp.where(ln >= FC, nf + 1, nf)
            ln = jnp.where(ln >= FC, ln - FC, ln)
        return (ln, nf)

    ln, nf = lax.fori_loop(0, NCHUNK, chunk, (jnp.int32(0), jnp.int32(0)))

    @pl.when(ln > 0)
    def _():
        lnc = ln
        junk16 = jnp.full((16,), JUNK, jnp.int32)
        zero16 = jnp.zeros((16,), jnp.int32)
        iota16 = lax.iota(jnp.int32, 16)
        for g in range(FC // 16):
            cur = slb_v[pl.ds(g * 16, 16)]
            curd = dsb_v[pl.ds(g * 16, 16)]
            valid = (g * 16 + iota16) < lnc
            slb_v[pl.ds(g * 16, 16)] = jnp.where(valid, cur, junk16)
            dsb_v[pl.ds(g * 16, 16)] = jnp.where(valid, curd, zero16)
        flush(nf)

    nf = jnp.where(ln > 0, nf + 1, nf)
    cnt_v[pl.ds(0, 16)] = jnp.broadcast_to(nf, (16,))
    pltpu.sync_copy(cnt_v, cnt_hbm.at[pl.ds(wid * 16, 16)])


def _acc1_body(slc_hbm, dstc_hbm, ev_hbm, cnt_hbm, h_hbm, hp_out, rs_out,
               slc_v, dstc_v, ev_v, cnt_v, rows_v, acc_v, gsem):
    cid = lax.axis_index("c")
    sid = lax.axis_index("s")
    lo, nr = _tile_bounds(sid)
    wid = cid * NS + sid
    obase = wid * CAP

    zero16 = jnp.zeros((16,), jnp.float32)

    def zrow(i, carry):
        for q in range(D // 16):
            acc_v[i, pl.ds(q * 16, 16)] = zero16
        return carry

    lax.fori_loop(0, ACC1, zrow, 0)

    pltpu.sync_copy(cnt_hbm.at[pl.ds(wid * 16, 16)], cnt_v)
    n_blocks = cnt_v[pl.ds(0, 16)][0]

    def block(k, carry):
        base = obase + k * FC
        pltpu.sync_copy(slc_hbm.at[pl.ds(base, FC)], slc_v)
        pltpu.sync_copy(dstc_hbm.at[pl.ds(base, FC)], dstc_v)
        pltpu.sync_copy(ev_hbm.at[pl.ds(base, FC)], ev_v)
        cps = [
            pltpu.async_copy(h_hbm.at[dstc_v.at[pl.ds(b * GSUB, GSUB)]],
                             rows_v.at[pl.ds(b * GSUB, GSUB)], gsem)
            for b in range(FC // GSUB)
        ]
        for cp in cps:
            cp.wait()

        def accum(g, carry2):
            sl16 = slc_v[pl.ds(g * 16, 16)]
            e16 = ev_v[pl.ds(g * 16, 16)]
            for l in range(16):
                sl = sl16[l]
                e = e16[l]
                i = g * 16 + l
                for q in range(D // 16):
                    acc_v[sl, pl.ds(q * 16, 16)] = (
                        acc_v[sl, pl.ds(q * 16, 16)]
                        + e * rows_v[i, pl.ds(q * 16, 16)])
                rrow = RPT_LAST + (sl >> 3)
                rcol = (sl & 7) * 16
                acc_v[rrow, pl.ds(rcol, 16)] = (
                    acc_v[rrow, pl.ds(rcol, 16)] + e)
            return carry2

        lax.fori_loop(0, FC // 16, accum, 0)
        return carry

    lax.fori_loop(0, n_blocks, block, 0)

    @pl.when(sid < NS - 1)
    def _():
        pltpu.sync_copy(acc_v.at[pl.ds(0, RPT)],
                        hp_out.at[pl.ds(cid * N + lo, RPT)])

    @pl.when(sid == NS - 1)
    def _():
        pltpu.sync_copy(acc_v.at[pl.ds(0, RPT_LAST)],
                        hp_out.at[pl.ds(cid * N + lo, RPT_LAST)])

    pltpu.sync_copy(acc_v.at[pl.ds(RPT_LAST, RSROWS)],
                    rs_out.at[pl.ds(wid * RSROWS, RSROWS)])


def _acc2_body(slc_hbm, dstc_hbm, cnt_hbm, m_hbm, si_out,
               slc_v, dstc_v, cnt_v, rows_v, acc_v, gsem):
    cid = lax.axis_index("c")
    sid = lax.axis_index("s")
    lo, nr = _tile_bounds(sid)
    wid = cid * NS + sid
    obase = wid * CAP

    zero16 = jnp.zeros((16,), jnp.float32)

    def zrow(i, carry):
        for q in range(D // 16):
            acc_v[i, pl.ds(q * 16, 16)] = zero16
        return carry

    lax.fori_loop(0, ACC2, zrow, 0)

    pltpu.sync_copy(cnt_hbm.at[pl.ds(wid * 16, 16)], cnt_v)
    n_blocks = cnt_v[pl.ds(0, 16)][0]

    def block(k, carry):
        base = obase + k * FC
        pltpu.sync_copy(slc_hbm.at[pl.ds(base, FC)], slc_v)
        pltpu.sync_copy(dstc_hbm.at[pl.ds(base, FC)], dstc_v)
        cps = [
            pltpu.async_copy(m_hbm.at[dstc_v.at[pl.ds(b * GSUB, GSUB)]],
                             rows_v.at[pl.ds(b * GSUB, GSUB)], gsem)
            for b in range(FC // GSUB)
        ]
        for cp in cps:
            cp.wait()

        def accum(g, carry2):
            sl16 = slc_v[pl.ds(g * 16, 16)]
            for l in range(16):
                sl = sl16[l]
                i = g * 16 + l
                for q in range(D // 16):
                    acc_v[sl, pl.ds(q * 16, 16)] = (
                        acc_v[sl, pl.ds(q * 16, 16)]
                        + rows_v[i, pl.ds(q * 16, 16)])
            return carry2

        lax.fori_loop(0, FC // 16, accum, 0)
        return carry

    lax.fori_loop(0, n_blocks, block, 0)

    @pl.when(sid < NS - 1)
    def _():
        pltpu.sync_copy(acc_v.at[pl.ds(0, RPT)],
                        si_out.at[pl.ds(cid * N + lo, RPT)])

    @pl.when(sid == NS - 1)
    def _():
        pltpu.sync_copy(acc_v.at[pl.ds(0, RPT_LAST)],
                        si_out.at[pl.ds(cid * N + lo, RPT_LAST)])


def _mid_body(hp_ref, rs_ref, x_ref, m_ref, se_ref):
    hp = hp_ref[0] + hp_ref[1]
    rs = rs_ref[...].sum(axis=1, keepdims=True)
    hprime = hp / (rs + 1e-16)
    m0 = jnp.where(hprime > 0.0, hprime, jnp.exp(hprime) - 1.0)
    mx = jnp.max(m0, axis=0, keepdims=True)
    z = jnp.exp(m0 - mx)
    sm = jnp.sum(z, axis=0, keepdims=True)
    m = z / sm
    m_ref[...] = m
    se_ref[...] = lax.dot_general(m, x_ref[...], (((0,), (0,)), ((), ())),
                                  preferred_element_type=jnp.float32)


def _post_body(si_ref, m_ref, sa_ref):
    si = si_ref[0] + si_ref[1]
    t = lax.dot_general(m_ref[...], si, (((0,), (0,)), ((), ())),
                        preferred_element_type=jnp.float32)
    sa_ref[...] = jnp.maximum(t - 1e-4, 0.0)


def kernel(main_feat, edge_index, W, a):
    f32 = jnp.float32
    i32 = jnp.int32
    src = edge_index[0]
    dst = edge_index[1]
    acols = a[0].reshape(2, D).T

    h, s = pl.pallas_call(
        _pre_body,
        out_shape=[jax.ShapeDtypeStruct((N, D), f32),
                   jax.ShapeDtypeStruct((N, 2), f32)],
    )(main_feat, W, acols)
    s1 = s[:, 0]
    s2 = s[:, 1]

    mesh = plsc.VectorSubcoreMesh(core_axis_name="c", subcore_axis_name="s")

    scan = pl.kernel(
        _scan_body,
        out_type=[jax.ShapeDtypeStruct((NW * CAP,), i32),
                  jax.ShapeDtypeStruct((NW * CAP,), i32),
                  jax.ShapeDtypeStruct((NW * CAP,), f32),
                  jax.ShapeDtypeStruct((NW * 16,), i32)],
        mesh=mesh,
        compiler_params=pltpu.CompilerParams(needs_layout_passes=False),
        scratch_types=[
            pltpu.VMEM((CS,), i32),
            pltpu.VMEM((CS,), i32),
            pltpu.VMEM((SPILL,), i32),
            pltpu.VMEM((SPILL,), i32),
            pltpu.VMEM((FC,), f32),
            pltpu.VMEM((RPT_LAST,), f32),
            pltpu.VMEM((N,), f32),
            pltpu.VMEM((16,), i32),
        ],
    )
    slc, dstc, ev, cnt = scan(src, dst, s1, s2)

    acc1 = pl.kernel(
        _acc1_body,
        out_type=[jax.ShapeDtypeStruct((NC * N, D), f32),
                  jax.ShapeDtypeStruct((NW * RSROWS, D), f32)],
        mesh=mesh,
        scratch_types=[
            pltpu.VMEM((FC,), i32),
            pltpu.VMEM((FC,), i32),
            pltpu.VMEM((FC,), f32),
            pltpu.VMEM((16,), i32),
            pltpu.VMEM((FC, D), f32),
            pltpu.VMEM((ACC1, D), f32),
            pltpu.SemaphoreType.DMA,
        ],
    )
    hp2, rs2 = acc1(slc, dstc, ev, cnt, h)
    hp = hp2.reshape(NC, N, D)

    rs4 = rs2.reshape(NC, NS, RSROWS * 8, 16)[:, :, :, 0]
    parts = [rs4[:, t, :RPT] for t in range(NS - 1)] + [rs4[:, NS - 1, :]]
    rs = jnp.concatenate(parts, axis=1).T

    m, struct_emb = pl.pallas_call(
        _mid_body,
        out_shape=[jax.ShapeDtypeStruct((N, D), f32),
                   jax.ShapeDtypeStruct((D, D), f32)],
    )(hp, rs, main_feat)

    acc2 = pl.kernel(
        _acc2_body,
        out_type=jax.ShapeDtypeStruct((NC * N, D), f32),
        mesh=mesh,
        scratch_types=[
            pltpu.VMEM((FC,), i32),
            pltpu.VMEM((FC,), i32),
            pltpu.VMEM((16,), i32),
            pltpu.VMEM((FC, D), f32),
            pltpu.VMEM((ACC2, D), f32),
            pltpu.SemaphoreType.DMA,
        ],
    )
    si2 = acc2(slc, dstc, cnt, m)
    si = si2.reshape(NC, N, D)

    struct_adj = pl.pallas_call(
        _post_body,
        out_shape=jax.ShapeDtypeStruct((D, D), f32),
    )(si, m)

    return (struct_emb, struct_adj, m)


# speedup vs baseline: 184.0113x; 1.0424x over previous
import functools

import jax
import jax.numpy as jnp
from jax import lax
from jax.experimental import pallas as pl
from jax.experimental.pallas import tpu as pltpu
from jax.experimental.pallas import tpu_sc as plsc

N = 10000
E = 320000
D = 128
ALPHA = 0.2

NC = 2
NS = 16
NW = NC * NS
EPC = E // NC
CS = 640
NCHUNK = EPC // CS
FC = 160
GSUB = 40
SPILL = CS + FC + 16
CAP = EPC + FC
RPT = 624
RPT_LAST = N - RPT * (NS - 1)
JUNK = 720
RSROWS = RPT_LAST // 8
ACC1 = 736
ACC2 = 728


def _pre_body(x_ref, w_ref, ac_ref, h_ref, s_ref):
    h = jnp.dot(x_ref[...], w_ref[...], preferred_element_type=jnp.float32)
    h_ref[...] = h
    s_ref[...] = jnp.dot(h, ac_ref[...], preferred_element_type=jnp.float32)


def _tile_bounds(sid):
    lo = sid * RPT
    nr = jnp.where(sid == NS - 1, RPT_LAST, RPT)
    return lo, nr


def _scan_body(src_hbm, dst_hbm, s1_hbm, s2_hbm, slc_hbm, dstc_hbm, ev_hbm,
               cnt_hbm, src_v, dst_v, slb_v, dsb_v, evb_v, s1loc_v, s2_v,
               cnt_v):
    cid = lax.axis_index("c")
    sid = lax.axis_index("s")
    lo, nr = _tile_bounds(sid)
    wid = cid * NS + sid
    obase = wid * CAP
    lo16 = jnp.broadcast_to(lo, (16,))
    hi16 = jnp.broadcast_to(lo + nr, (16,))

    pltpu.sync_copy(s2_hbm, s2_v)
    pltpu.sync_copy(s1_hbm.at[pl.ds(lo, RPT_LAST)], s1loc_v)

    def flush(nf):
        for g in range(FC // 16):
            sl16 = jnp.minimum(slb_v[pl.ds(g * 16, 16)], RPT_LAST - 1)
            d16 = dsb_v[pl.ds(g * 16, 16)]
            sval = plsc.load_gather(s1loc_v, [sl16])
            dval = plsc.load_gather(s2_v, [d16])
            t = sval + dval
            lr = jnp.where(t > 0.0, t, ALPHA * t)
            evb_v[pl.ds(g * 16, 16)] = jnp.exp(-lr)
        out = obase + nf * FC
        pltpu.sync_copy(slb_v.at[pl.ds(0, FC)], slc_hbm.at[pl.ds(out, FC)])
        pltpu.sync_copy(dsb_v.at[pl.ds(0, FC)], dstc_hbm.at[pl.ds(out, FC)])
        pltpu.sync_copy(evb_v.at[pl.ds(0, FC)], ev_hbm.at[pl.ds(out, FC)])
        for g in range((SPILL - FC) // 16):
            slb_v[pl.ds(g * 16, 16)] = slb_v[pl.ds(FC + g * 16, 16)]
            dsb_v[pl.ds(g * 16, 16)] = dsb_v[pl.ds(FC + g * 16, 16)]

    def chunk(k, carry):
        ln, nf = carry
        base = cid * EPC + k * CS
        pltpu.sync_copy(src_hbm.at[pl.ds(base, CS)], src_v)
        pltpu.sync_copy(dst_hbm.at[pl.ds(base, CS)], dst_v)

        for j in range(CS // 16):
            s16 = src_v[pl.ds(j * 16, 16)]
            d16 = dst_v[pl.ds(j * 16, 16)]
            mask = jnp.logical_and(s16 >= lo16, s16 < hi16)
            plsc.store_compressed(slb_v.at[pl.ds(ln, 16)], s16 - lo16,
                                  mask=mask)
            plsc.store_compressed(dsb_v.at[pl.ds(ln, 16)], d16, mask=mask)
            ln = ln + jnp.sum(mask.astype(jnp.int32))

        for _ in range(CS // FC):
            @pl.when(ln >= FC)
            def _():
                flush(nf)

            nf = jnp.where(ln >= FC, nf + 1, nf)
            ln = jnp.where(ln >= FC, ln - FC, ln)
        return (ln, nf)

    ln, nf = lax.fori_loop(0, NCHUNK, chunk, (jnp.int32(0), jnp.int32(0)))

    @pl.when(ln > 0)
    def _():
        lnc = ln
        junk16 = jnp.full((16,), JUNK, jnp.int32)
        zero16 = jnp.zeros((16,), jnp.int32)
        iota16 = lax.iota(jnp.int32, 16)
        for g in range(FC // 16):
            cur = slb_v[pl.ds(g * 16, 16)]
            curd = dsb_v[pl.ds(g * 16, 16)]
            valid = (g * 16 + iota16) < lnc
            slb_v[pl.ds(g * 16, 16)] = jnp.where(valid, cur, junk16)
            dsb_v[pl.ds(g * 16, 16)] = jnp.where(valid, curd, zero16)
        flush(nf)

    nf = jnp.where(ln > 0, nf + 1, nf)
    cnt_v[pl.ds(0, 16)] = jnp.broadcast_to(nf, (16,))
    pltpu.sync_copy(cnt_v, cnt_hbm.at[pl.ds(wid * 16, 16)])


def _acc1_body(slc_hbm, dstc_hbm, ev_hbm, cnt_hbm, h_hbm, hp_out, rs_out,
               slc_v, dstc_v, ev_v, cnt_v, rows_v, acc_v, gsem):
    cid = lax.axis_index("c")
    sid = lax.axis_index("s")
    lo, nr = _tile_bounds(sid)
    wid = cid * NS + sid
    obase = wid * CAP

    zero16 = jnp.zeros((16,), jnp.float32)

    def zrow(i, carry):
        for q in range(D // 16):
            acc_v[i, pl.ds(q * 16, 16)] = zero16
        return carry

    lax.fori_loop(0, ACC1, zrow, 0)

    pltpu.sync_copy(cnt_hbm.at[pl.ds(wid * 16, 16)], cnt_v)
    n_blocks = cnt_v[pl.ds(0, 16)][0]

    def block(k, carry):
        base = obase + k * FC
        cpd = pltpu.async_copy(dstc_hbm.at[pl.ds(base, FC)], dstc_v, gsem)
        cpl = pltpu.async_copy(slc_hbm.at[pl.ds(base, FC)], slc_v, gsem)
        cpe = pltpu.async_copy(ev_hbm.at[pl.ds(base, FC)], ev_v, gsem)
        cpd.wait()
        cps = [
            pltpu.async_copy(h_hbm.at[dstc_v.at[pl.ds(b * GSUB, GSUB)]],
                             rows_v.at[pl.ds(b * GSUB, GSUB)], gsem)
            for b in range(FC // GSUB)
        ]
        cpl.wait()
        cpe.wait()
        for cp in cps:
            cp.wait()

        def accum(g, carry2):
            sl16 = slc_v[pl.ds(g * 16, 16)]
            e16 = ev_v[pl.ds(g * 16, 16)]
            for l in range(16):
                sl = sl16[l]
                e = e16[l]
                i = g * 16 + l
                for q in range(D // 16):
                    acc_v[sl, pl.ds(q * 16, 16)] = (
                        acc_v[sl, pl.ds(q * 16, 16)]
                        + e * rows_v[i, pl.ds(q * 16, 16)])
                rrow = RPT_LAST + (sl >> 3)
                rcol = (sl & 7) * 16
                acc_v[rrow, pl.ds(rcol, 16)] = (
                    acc_v[rrow, pl.ds(rcol, 16)] + e)
            return carry2

        lax.fori_loop(0, FC // 16, accum, 0)
        return carry

    lax.fori_loop(0, n_blocks, block, 0)

    @pl.when(sid < NS - 1)
    def _():
        pltpu.sync_copy(acc_v.at[pl.ds(0, RPT)],
                        hp_out.at[pl.ds(cid * N + lo, RPT)])

    @pl.when(sid == NS - 1)
    def _():
        pltpu.sync_copy(acc_v.at[pl.ds(0, RPT_LAST)],
                        hp_out.at[pl.ds(cid * N + lo, RPT_LAST)])

    pltpu.sync_copy(acc_v.at[pl.ds(RPT_LAST, RSROWS)],
                    rs_out.at[pl.ds(wid * RSROWS, RSROWS)])


def _acc2_body(slc_hbm, dstc_hbm, cnt_hbm, m_hbm, si_out,
               slc_v, dstc_v, cnt_v, rows_v, acc_v, gsem):
    cid = lax.axis_index("c")
    sid = lax.axis_index("s")
    lo, nr = _tile_bounds(sid)
    wid = cid * NS + sid
    obase = wid * CAP

    zero16 = jnp.zeros((16,), jnp.float32)

    def zrow(i, carry):
        for q in range(D // 16):
            acc_v[i, pl.ds(q * 16, 16)] = zero16
        return carry

    lax.fori_loop(0, ACC2, zrow, 0)

    pltpu.sync_copy(cnt_hbm.at[pl.ds(wid * 16, 16)], cnt_v)
    n_blocks = cnt_v[pl.ds(0, 16)][0]

    def block(k, carry):
        base = obase + k * FC
        cpd = pltpu.async_copy(dstc_hbm.at[pl.ds(base, FC)], dstc_v, gsem)
        cpl = pltpu.async_copy(slc_hbm.at[pl.ds(base, FC)], slc_v, gsem)
        cpd.wait()
        cps = [
            pltpu.async_copy(m_hbm.at[dstc_v.at[pl.ds(b * GSUB, GSUB)]],
                             rows_v.at[pl.ds(b * GSUB, GSUB)], gsem)
            for b in range(FC // GSUB)
        ]
        cpl.wait()
        for cp in cps:
            cp.wait()

        def accum(g, carry2):
            sl16 = slc_v[pl.ds(g * 16, 16)]
            for l in range(16):
                sl = sl16[l]
                i = g * 16 + l
                for q in range(D // 16):
                    acc_v[sl, pl.ds(q * 16, 16)] = (
                        acc_v[sl, pl.ds(q * 16, 16)]
                        + rows_v[i, pl.ds(q * 16, 16)])
            return carry2

        lax.fori_loop(0, FC // 16, accum, 0)
        return carry

    lax.fori_loop(0, n_blocks, block, 0)

    @pl.when(sid < NS - 1)
    def _():
        pltpu.sync_copy(acc_v.at[pl.ds(0, RPT)],
                        si_out.at[pl.ds(cid * N + lo, RPT)])

    @pl.when(sid == NS - 1)
    def _():
        pltpu.sync_copy(acc_v.at[pl.ds(0, RPT_LAST)],
                        si_out.at[pl.ds(cid * N + lo, RPT_LAST)])


def _mid_body(hp_ref, rs_ref, x_ref, m_ref, se_ref):
    hp = hp_ref[0] + hp_ref[1]
    rs = rs_ref[...].sum(axis=1, keepdims=True)
    hprime = hp / (rs + 1e-16)
    m0 = jnp.where(hprime > 0.0, hprime, jnp.exp(hprime) - 1.0)
    mx = jnp.max(m0, axis=0, keepdims=True)
    z = jnp.exp(m0 - mx)
    sm = jnp.sum(z, axis=0, keepdims=True)
    m = z / sm
    m_ref[...] = m
    se_ref[...] = lax.dot_general(m, x_ref[...], (((0,), (0,)), ((), ())),
                                  preferred_element_type=jnp.float32)


def _post_body(si_ref, m_ref, sa_ref):
    si = si_ref[0] + si_ref[1]
    t = lax.dot_general(m_ref[...], si, (((0,), (0,)), ((), ())),
                        preferred_element_type=jnp.float32)
    sa_ref[...] = jnp.maximum(t - 1e-4, 0.0)


def kernel(main_feat, edge_index, W, a):
    f32 = jnp.float32
    i32 = jnp.int32
    src = edge_index[0]
    dst = edge_index[1]
    acols = a[0].reshape(2, D).T

    h, s = pl.pallas_call(
        _pre_body,
        out_shape=[jax.ShapeDtypeStruct((N, D), f32),
                   jax.ShapeDtypeStruct((N, 2), f32)],
    )(main_feat, W, acols)
    s1 = s[:, 0]
    s2 = s[:, 1]

    mesh = plsc.VectorSubcoreMesh(core_axis_name="c", subcore_axis_name="s")

    scan = pl.kernel(
        _scan_body,
        out_type=[jax.ShapeDtypeStruct((NW * CAP,), i32),
                  jax.ShapeDtypeStruct((NW * CAP,), i32),
                  jax.ShapeDtypeStruct((NW * CAP,), f32),
                  jax.ShapeDtypeStruct((NW * 16,), i32)],
        mesh=mesh,
        compiler_params=pltpu.CompilerParams(needs_layout_passes=False),
        scratch_types=[
            pltpu.VMEM((CS,), i32),
            pltpu.VMEM((CS,), i32),
            pltpu.VMEM((SPILL,), i32),
            pltpu.VMEM((SPILL,), i32),
            pltpu.VMEM((FC,), f32),
            pltpu.VMEM((RPT_LAST,), f32),
            pltpu.VMEM((N,), f32),
            pltpu.VMEM((16,), i32),
        ],
    )
    slc, dstc, ev, cnt = scan(src, dst, s1, s2)

    acc1 = pl.kernel(
        _acc1_body,
        out_type=[jax.ShapeDtypeStruct((NC * N, D), f32),
                  jax.ShapeDtypeStruct((NW * RSROWS, D), f32)],
        mesh=mesh,
        scratch_types=[
            pltpu.VMEM((FC,), i32),
            pltpu.VMEM((FC,), i32),
            pltpu.VMEM((FC,), f32),
            pltpu.VMEM((16,), i32),
            pltpu.VMEM((FC, D), f32),
            pltpu.VMEM((ACC1, D), f32),
            pltpu.SemaphoreType.DMA,
        ],
    )
    hp2, rs2 = acc1(slc, dstc, ev, cnt, h)
    hp = hp2.reshape(NC, N, D)

    rs4 = rs2.reshape(NC, NS, RSROWS * 8, 16)[:, :, :, 0]
    parts = [rs4[:, t, :RPT] for t in range(NS - 1)] + [rs4[:, NS - 1, :]]
    rs = jnp.concatenate(parts, axis=1).T

    m, struct_emb = pl.pallas_call(
        _mid_body,
        out_shape=[jax.ShapeDtypeStruct((N, D), f32),
                   jax.ShapeDtypeStruct((D, D), f32)],
    )(hp, rs, main_feat)

    acc2 = pl.kernel(
        _acc2_body,
        out_type=jax.ShapeDtypeStruct((NC * N, D), f32),
        mesh=mesh,
        scratch_types=[
            pltpu.VMEM((FC,), i32),
            pltpu.VMEM((FC,), i32),
            pltpu.VMEM((16,), i32),
            pltpu.VMEM((FC, D), f32),
            pltpu.VMEM((ACC2, D), f32),
            pltpu.SemaphoreType.DMA,
        ],
    )
    si2 = acc2(slc, dstc, cnt, m)
    si = si2.reshape(NC, N, D)

    struct_adj = pl.pallas_call(
        _post_body,
        out_shape=jax.ShapeDtypeStruct((D, D), f32),
    )(si, m)

    return (struct_emb, struct_adj, m)


# speedup vs baseline: 190.3624x; 1.0345x over previous
import functools

import jax
import jax.numpy as jnp
from jax import lax
from jax.experimental import pallas as pl
from jax.experimental.pallas import tpu as pltpu
from jax.experimental.pallas import tpu_sc as plsc

N = 10000
E = 320000
D = 128
ALPHA = 0.2

NC = 2
NS = 16
NW = NC * NS
EPC = E // NC
CS = 640
NCHUNK = EPC // CS
FC = 160
GSUB = 40
SPILL = CS + FC + 16
CAP = EPC + FC
RPT = 624
RPT_LAST = N - RPT * (NS - 1)
JUNK = 720
RSROWS = RPT_LAST // 8
ACC1 = 736
ACC2 = 648


def _pre_body(x_ref, w_ref, ac_ref, h_ref, s_ref):
    h = jnp.dot(x_ref[...], w_ref[...], preferred_element_type=jnp.float32)
    h_ref[...] = h
    s_ref[...] = jnp.dot(h, ac_ref[...], preferred_element_type=jnp.float32)


def _tile_bounds(sid):
    lo = sid * RPT
    nr = jnp.where(sid == NS - 1, RPT_LAST, RPT)
    return lo, nr


def _scan_body(src_hbm, dst_hbm, s1_hbm, s2_hbm, slc_hbm, dstc_hbm, ev_hbm,
               cnt_hbm, src_v, dst_v, slb_v, dsb_v, evb_v, s1loc_v, s2_v,
               cnt_v):
    cid = lax.axis_index("c")
    sid = lax.axis_index("s")
    lo, nr = _tile_bounds(sid)
    wid = cid * NS + sid
    obase = wid * CAP
    lo16 = jnp.broadcast_to(lo, (16,))
    hi16 = jnp.broadcast_to(lo + nr, (16,))

    pltpu.sync_copy(s2_hbm, s2_v)
    pltpu.sync_copy(s1_hbm.at[pl.ds(lo, RPT_LAST)], s1loc_v)

    def flush(nf):
        for g in range(FC // 16):
            sl16 = jnp.minimum(slb_v[pl.ds(g * 16, 16)], RPT_LAST - 1)
            d16 = dsb_v[pl.ds(g * 16, 16)]
            sval = plsc.load_gather(s1loc_v, [sl16])
            dval = plsc.load_gather(s2_v, [d16])
            t = sval + dval
            lr = jnp.where(t > 0.0, t, ALPHA * t)
            evb_v[pl.ds(g * 16, 16)] = jnp.exp(-lr)
        out = obase + nf * FC
        pltpu.sync_copy(slb_v.at[pl.ds(0, FC)], slc_hbm.at[pl.ds(out, FC)])
        pltpu.sync_copy(dsb_v.at[pl.ds(0, FC)], dstc_hbm.at[pl.ds(out, FC)])
        pltpu.sync_copy(evb_v.at[pl.ds(0, FC)], ev_hbm.at[pl.ds(out, FC)])
        for g in range((SPILL - FC) // 16):
            slb_v[pl.ds(g * 16, 16)] = slb_v[pl.ds(FC + g * 16, 16)]
            dsb_v[pl.ds(g * 16, 16)] = dsb_v[pl.ds(FC + g * 16, 16)]

    def chunk(k, carry):
        ln, nf = carry
        base = cid * EPC + k * CS
        pltpu.sync_copy(src_hbm.at[pl.ds(base, CS)], src_v)
        pltpu.sync_copy(dst_hbm.at[pl.ds(base, CS)], dst_v)

        for j in range(CS // 16):
            s16 = src_v[pl.ds(j * 16, 16)]
            d16 = dst_v[pl.ds(j * 16, 16)]
            mask = jnp.logical_and(s16 >= lo16, s16 < hi16)
            plsc.store_compressed(slb_v.at[pl.ds(ln, 16)], s16 - lo16,
                                  mask=mask)
            plsc.store_compressed(dsb_v.at[pl.ds(ln, 16)], d16, mask=mask)
            ln = ln + jnp.sum(mask.astype(jnp.int32))

        for _ in range(CS // FC):
            @pl.when(ln >= FC)
            def _():
                flush(nf)

            nf = jnp.where(ln >= FC, nf + 1, nf)
            ln = jnp.where(ln >= FC, ln - FC, ln)
        return (ln, nf)

    ln, nf = lax.fori_loop(0, NCHUNK, chunk, (jnp.int32(0), jnp.int32(0)))

    @pl.when(ln > 0)
    def _():
        lnc = ln
        junk16 = jnp.full((16,), JUNK, jnp.int32)
        zero16 = jnp.zeros((16,), jnp.int32)
        iota16 = lax.iota(jnp.int32, 16)
        for g in range(FC // 16):
            cur = slb_v[pl.ds(g * 16, 16)]
            curd = dsb_v[pl.ds(g * 16, 16)]
            valid = (g * 16 + iota16) < lnc
            slb_v[pl.ds(g * 16, 16)] = jnp.where(valid, cur, junk16)
            dsb_v[pl.ds(g * 16, 16)] = jnp.where(valid, curd, zero16)
        flush(nf)

    nf = jnp.where(ln > 0, nf + 1, nf)
    cnt_v[pl.ds(0, 16)] = jnp.broadcast_to(nf, (16,))
    pltpu.sync_copy(cnt_v, cnt_hbm.at[pl.ds(wid * 16, 16)])


def _acc1_body(slc_hbm, dstc_hbm, ev_hbm, cnt_hbm, h_hbm, hp_out, rs_out,
               slc_v, dstc_v, ev_v, cnt_v, rows_v, acc_v, gsem):
    cid = lax.axis_index("c")
    sid = lax.axis_index("s")
    lo, nr = _tile_bounds(sid)
    wid = cid * NS + sid
    obase = wid * CAP

    zero16 = jnp.zeros((16,), jnp.float32)

    def zrow(i, carry):
        for q in range(D // 16):
            acc_v[i, pl.ds(q * 16, 16)] = zero16
        return carry

    lax.fori_loop(0, ACC1, zrow, 0)

    pltpu.sync_copy(cnt_hbm.at[pl.ds(wid * 16, 16)], cnt_v)
    n_blocks = cnt_v[pl.ds(0, 16)][0]

    def block(k, carry):
        base = obase + k * FC
        cpd = pltpu.async_copy(dstc_hbm.at[pl.ds(base, FC)], dstc_v, gsem)
        cpl = pltpu.async_copy(slc_hbm.at[pl.ds(base, FC)], slc_v, gsem)
        cpe = pltpu.async_copy(ev_hbm.at[pl.ds(base, FC)], ev_v, gsem)
        cpd.wait()
        cps = [
            pltpu.async_copy(h_hbm.at[dstc_v.at[pl.ds(b * GSUB, GSUB)]],
                             rows_v.at[pl.ds(b * GSUB, GSUB)], gsem)
            for b in range(FC // GSUB)
        ]
        cpl.wait()
        cpe.wait()
        for cp in cps:
            cp.wait()

        def accum(g, carry2):
            sl16 = slc_v[pl.ds(g * 16, 16)]
            e16 = ev_v[pl.ds(g * 16, 16)]
            for l in range(16):
                sl = sl16[l]
                e = e16[l]
                i = g * 16 + l
                for q in range(D // 16):
                    acc_v[sl, pl.ds(q * 16, 16)] = (
                        acc_v[sl, pl.ds(q * 16, 16)]
                        + e * rows_v[i, pl.ds(q * 16, 16)])
                rrow = RPT_LAST + (sl >> 3)
                rcol = (sl & 7) * 16
                acc_v[rrow, pl.ds(rcol, 16)] = (
                    acc_v[rrow, pl.ds(rcol, 16)] + e)
            return carry2

        lax.fori_loop(0, FC // 16, accum, 0)
        return carry

    lax.fori_loop(0, n_blocks, block, 0)

    @pl.when(sid < NS - 1)
    def _():
        pltpu.sync_copy(acc_v.at[pl.ds(0, RPT)],
                        hp_out.at[pl.ds(cid * N + lo, RPT)])

    @pl.when(sid == NS - 1)
    def _():
        pltpu.sync_copy(acc_v.at[pl.ds(0, RPT_LAST)],
                        hp_out.at[pl.ds(cid * N + lo, RPT_LAST)])

    pltpu.sync_copy(acc_v.at[pl.ds(RPT_LAST, RSROWS)],
                    rs_out.at[pl.ds(wid * RSROWS, RSROWS)])


def _acc2_body(slc_hbm, dstc_hbm, cnt_hbm, m_hbm, si_out,
               slc_v, dstc_v, cnt_v, rows_v, acc_v, gsem0, gsem1):
    cid = lax.axis_index("c")
    sid = lax.axis_index("s")
    lo, nr = _tile_bounds(sid)
    wid = cid * NS + sid
    obase = wid * CAP

    zero16 = jnp.zeros((16,), jnp.float32)

    def zrow(i, carry):
        for q in range(D // 16):
            acc_v[i, pl.ds(q * 16, 16)] = zero16
        return carry

    lax.fori_loop(0, ACC2, zrow, 0)

    pltpu.sync_copy(cnt_hbm.at[pl.ds(wid * 16, 16)], cnt_v)
    n_blocks = cnt_v[pl.ds(0, 16)][0]

    def issue_gathers(k, p, sem):
        base = obase + k * FC
        pltpu.sync_copy(slc_hbm.at[pl.ds(base, FC)],
                        slc_v.at[pl.ds(p * FC, FC)])
        pltpu.sync_copy(dstc_hbm.at[pl.ds(base, FC)],
                        dstc_v.at[pl.ds(p * FC, FC)])
        for b in range(FC // GSUB):
            pltpu.async_copy(
                m_hbm.at[dstc_v.at[pl.ds(p * FC + b * GSUB, GSUB)]],
                rows_v.at[pl.ds(p * FC + b * GSUB, GSUB)], sem)

    def wait_gathers(k, p, sem):
        for b in range(FC // GSUB):
            pltpu.make_async_copy(
                m_hbm.at[dstc_v.at[pl.ds(p * FC + b * GSUB, GSUB)]],
                rows_v.at[pl.ds(p * FC + b * GSUB, GSUB)], sem).wait()

    @pl.when(n_blocks > 0)
    def _():
        issue_gathers(0, 0, gsem0)

    def block(k, carry):
        p = jnp.bitwise_and(k, 1)

        @pl.when(k + 1 < n_blocks)
        def _():
            @pl.when(p == 0)
            def _():
                issue_gathers(k + 1, 1, gsem1)

            @pl.when(p == 1)
            def _():
                issue_gathers(k + 1, 0, gsem0)

        @pl.when(p == 0)
        def _():
            wait_gathers(k, 0, gsem0)

        @pl.when(p == 1)
        def _():
            wait_gathers(k, 1, gsem1)

        def accum(g, carry2):
            sl16 = jnp.minimum(slc_v[pl.ds(p * FC + g * 16, 16)], ACC2 - 1)
            for l in range(16):
                sl = sl16[l]
                i = g * 16 + l
                for q in range(D // 16):
                    acc_v[sl, pl.ds(q * 16, 16)] = (
                        acc_v[sl, pl.ds(q * 16, 16)]
                        + rows_v[p * FC + i, pl.ds(q * 16, 16)])
            return carry2

        lax.fori_loop(0, FC // 16, accum, 0)
        return carry

    lax.fori_loop(0, n_blocks, block, 0)

    @pl.when(sid < NS - 1)
    def _():
        pltpu.sync_copy(acc_v.at[pl.ds(0, RPT)],
                        si_out.at[pl.ds(cid * N + lo, RPT)])

    @pl.when(sid == NS - 1)
    def _():
        pltpu.sync_copy(acc_v.at[pl.ds(0, RPT_LAST)],
                        si_out.at[pl.ds(cid * N + lo, RPT_LAST)])


def _mid_body(hp_ref, rs_ref, x_ref, m_ref, se_ref):
    hp = hp_ref[0] + hp_ref[1]
    rs = rs_ref[...].sum(axis=1, keepdims=True)
    hprime = hp / (rs + 1e-16)
    m0 = jnp.where(hprime > 0.0, hprime, jnp.exp(hprime) - 1.0)
    mx = jnp.max(m0, axis=0, keepdims=True)
    z = jnp.exp(m0 - mx)
    sm = jnp.sum(z, axis=0, keepdims=True)
    m = z / sm
    m_ref[...] = m
    se_ref[...] = lax.dot_general(m, x_ref[...], (((0,), (0,)), ((), ())),
                                  preferred_element_type=jnp.float32)


def _post_body(si_ref, m_ref, sa_ref):
    si = si_ref[0] + si_ref[1]
    t = lax.dot_general(m_ref[...], si, (((0,), (0,)), ((), ())),
                        preferred_element_type=jnp.float32)
    sa_ref[...] = jnp.maximum(t - 1e-4, 0.0)


def kernel(main_feat, edge_index, W, a):
    f32 = jnp.float32
    i32 = jnp.int32
    src = edge_index[0]
    dst = edge_index[1]
    acols = a[0].reshape(2, D).T

    h, s = pl.pallas_call(
        _pre_body,
        out_shape=[jax.ShapeDtypeStruct((N, D), f32),
                   jax.ShapeDtypeStruct((N, 2), f32)],
    )(main_feat, W, acols)
    s1 = s[:, 0]
    s2 = s[:, 1]

    mesh = plsc.VectorSubcoreMesh(core_axis_name="c", subcore_axis_name="s")

    scan = pl.kernel(
        _scan_body,
        out_type=[jax.ShapeDtypeStruct((NW * CAP,), i32),
                  jax.ShapeDtypeStruct((NW * CAP,), i32),
                  jax.ShapeDtypeStruct((NW * CAP,), f32),
                  jax.ShapeDtypeStruct((NW * 16,), i32)],
        mesh=mesh,
        compiler_params=pltpu.CompilerParams(needs_layout_passes=False),
        scratch_types=[
            pltpu.VMEM((CS,), i32),
            pltpu.VMEM((CS,), i32),
            pltpu.VMEM((SPILL,), i32),
            pltpu.VMEM((SPILL,), i32),
            pltpu.VMEM((FC,), f32),
            pltpu.VMEM((RPT_LAST,), f32),
            pltpu.VMEM((N,), f32),
            pltpu.VMEM((16,), i32),
        ],
    )
    slc, dstc, ev, cnt = scan(src, dst, s1, s2)

    acc1 = pl.kernel(
        _acc1_body,
        out_type=[jax.ShapeDtypeStruct((NC * N, D), f32),
                  jax.ShapeDtypeStruct((NW * RSROWS, D), f32)],
        mesh=mesh,
        scratch_types=[
            pltpu.VMEM((FC,), i32),
            pltpu.VMEM((FC,), i32),
            pltpu.VMEM((FC,), f32),
            pltpu.VMEM((16,), i32),
            pltpu.VMEM((FC, D), f32),
            pltpu.VMEM((ACC1, D), f32),
            pltpu.SemaphoreType.DMA,
        ],
    )
    hp2, rs2 = acc1(slc, dstc, ev, cnt, h)
    hp = hp2.reshape(NC, N, D)

    rs4 = rs2.reshape(NC, NS, RSROWS * 8, 16)[:, :, :, 0]
    parts = [rs4[:, t, :RPT] for t in range(NS - 1)] + [rs4[:, NS - 1, :]]
    rs = jnp.concatenate(parts, axis=1).T

    m, struct_emb = pl.pallas_call(
        _mid_body,
        out_shape=[jax.ShapeDtypeStruct((N, D), f32),
                   jax.ShapeDtypeStruct((D, D), f32)],
    )(hp, rs, main_feat)

    acc2 = pl.kernel(
        _acc2_body,
        out_type=jax.ShapeDtypeStruct((NC * N, D), f32),
        mesh=mesh,
        scratch_types=[
            pltpu.VMEM((2 * FC,), i32),
            pltpu.VMEM((2 * FC,), i32),
            pltpu.VMEM((16,), i32),
            pltpu.VMEM((2 * FC, D), f32),
            pltpu.VMEM((ACC2, D), f32),
            pltpu.SemaphoreType.DMA,
            pltpu.SemaphoreType.DMA,
        ],
    )
    si2 = acc2(slc, dstc, cnt, m)
    si = si2.reshape(NC, N, D)

    struct_adj = pl.pallas_call(
        _post_body,
        out_shape=jax.ShapeDtypeStruct((D, D), f32),
    )(si, m)

    return (struct_emb, struct_adj, m)
